# baseline (speedup 1.0000x reference)
import jax
import jax.numpy as jnp
from jax import lax
from jax.experimental import pallas as pl
from jax.experimental.pallas import tpu as pltpu
from jax.experimental.pallas import tpu_sc as plsc

BATCH = 16384
DIM = 16
PAIRS = DIM // 2

_NC = 2
_NS = 16
_NW = _NC * _NS
_ROWS_PER_W = BATCH // _NW
_EL_PER_W = _ROWS_PER_W * PAIRS
_CHUNK = 128
_NCHUNK = _EL_PER_W // _CHUNK
_NGROUP = _ROWS_PER_W // 16
_WAVE = 16


def _tt_body(x_hbm, y_hbm, art_hbm, cust_hbm, out_hbm,
             xidx, yidx, xe, ye, xrows, yrows, out_v, sem):
    wid = lax.axis_index("s") * _NC + lax.axis_index("c")
    base = wid * _ROWS_PER_W

    cp_x = pltpu.make_async_copy(x_hbm.at[pl.ds(base, _ROWS_PER_W)], xidx, sem)
    cp_y = pltpu.make_async_copy(y_hbm.at[pl.ds(base, _ROWS_PER_W)], yidx, sem)
    cp_x.start()
    cp_y.start()
    cp_x.wait()
    cp_y.wait()

    iota = lax.iota(jnp.int32, 16)
    p8 = lax.bitwise_and(iota, 7)
    hi8 = iota >= 8

    def expand(g, carry):
        vx = xidx[pl.ds(g * 16, 16)]
        vy = yidx[pl.ds(g * 16, 16)]
        for k in range(8):
            b2 = g * 16 + 2 * k
            sx = jnp.where(hi8, jnp.full((16,), vx[2 * k + 1], jnp.int32),
                           jnp.full((16,), vx[2 * k], jnp.int32))
            sy = jnp.where(hi8, jnp.full((16,), vy[2 * k + 1], jnp.int32),
                           jnp.full((16,), vy[2 * k], jnp.int32))
            xe[pl.ds(b2 * 8, 16)] = sx * 8 + p8
            ye[pl.ds(b2 * 8, 16)] = sy * 8 + p8
        return carry

    lax.fori_loop(0, _NGROUP, expand, None)

    copies = []
    for j in range(_NCHUNK):
        sl = pl.ds(j * _CHUNK, _CHUNK)
        copies.append(pltpu.make_async_copy(cust_hbm.at[xe.at[sl]], xrows.at[sl], sem))
        copies.append(pltpu.make_async_copy(art_hbm.at[ye.at[sl]], yrows.at[sl], sem))
    for w in range(0, len(copies), _WAVE):
        wave = copies[w:w + _WAVE]
        for c in wave:
            c.start()
        for c in wave:
            c.wait()

    def group(g, carry):
        fb = g * 128 + iota * PAIRS
        acc = jnp.zeros((16,), jnp.float32)
        for p in range(PAIRS):
            flat = fb + lax.bitwise_and(iota + p, 7)
            xw = plsc.load_gather(xrows, [flat])
            yw = plsc.load_gather(yrows, [flat])
            xa, xb = plsc.unpack(plsc.bitcast(xw, jnp.bfloat16),
                                 format=plsc.PackFormat.INTERLEAVED)
            ya, yb = plsc.unpack(plsc.bitcast(yw, jnp.bfloat16),
                                 format=plsc.PackFormat.INTERLEAVED)
            acc = acc + xa * ya + xb * yb
        out_v[pl.ds(g * 16, 16)] = acc
        return carry

    lax.fori_loop(0, _NGROUP, group, None)

    pltpu.sync_copy(out_v, out_hbm.at[pl.ds(base, _ROWS_PER_W)])


def _pack_rows(table):
    tb = table.astype(jnp.bfloat16)
    pairs = lax.bitcast_convert_type(tb.reshape(-1, PAIRS, 2), jnp.int32)
    return pairs.reshape(-1)


def kernel(x, y, article_table, customer_table):
    x = x.astype(jnp.int32)
    y = y.astype(jnp.int32)
    art_p = _pack_rows(article_table)
    cust_p = _pack_rows(customer_table)
    mesh = plsc.VectorSubcoreMesh(
        core_axis_name="c", subcore_axis_name="s",
        num_cores=_NC, num_subcores=_NS)
    run = pl.kernel(
        _tt_body,
        out_type=jax.ShapeDtypeStruct((BATCH,), jnp.float32),
        mesh=mesh,
        scratch_types=[
            pltpu.VMEM((_ROWS_PER_W,), jnp.int32),
            pltpu.VMEM((_ROWS_PER_W,), jnp.int32),
            pltpu.VMEM((_EL_PER_W,), jnp.int32),
            pltpu.VMEM((_EL_PER_W,), jnp.int32),
            pltpu.VMEM((_EL_PER_W,), jnp.int32),
            pltpu.VMEM((_EL_PER_W,), jnp.int32),
            pltpu.VMEM((_ROWS_PER_W,), jnp.float32),
            pltpu.SemaphoreType.DMA,
        ],
        compiler_params=pltpu.CompilerParams(
            needs_layout_passes=False, use_tc_tiling_on_sc=False),
    )
    return run(x, y, art_p, cust_p)
